# baseline (speedup 1.0000x reference)
import functools

import jax
import jax.numpy as jnp
from jax import lax
from jax.experimental import pallas as pl
from jax.experimental.pallas import tpu as pltpu
from jax.experimental.pallas import tpu_sc as plsc

_H = 128
_NMOL = 1024
_N_PAD = 114688
_BLK = 57344
_ROWS = _N_PAD // 128
_W = 16
_RPW = _ROWS // _W


def _tc_body(n_atoms, z_ref, posT_ref, tabT_ref, wo_ref, y_ref):
    zb = z_ref[...][None, :]
    row = lax.broadcasted_iota(jnp.int32, (_H, _BLK), 0)
    ohT = (zb == row).astype(jnp.bfloat16)
    rhs = jnp.concatenate([ohT, posT_ref[...]], axis=0)
    xT = jnp.dot(tabT_ref[...], rhs, preferred_element_type=jnp.float32)
    yT = jnp.sum(jnp.tanh(xT) * wo_ref[...], axis=0,
                 keepdims=True)
    gidx = (pl.program_id(0) * _BLK
            + lax.broadcasted_iota(jnp.int32, (1, _BLK), 1))
    y_ref[0] = jnp.where(gidx < n_atoms, yT, 0.0)


def _tc_stage(z1, posT8, tabT2, wo):
    grid = (_N_PAD // _BLK,)
    return pl.pallas_call(
        functools.partial(_tc_body, z1.shape[0]),
        grid=grid,
        in_specs=[
            pl.BlockSpec((_BLK,), lambda i: (i,)),
            pl.BlockSpec((8, _BLK), lambda i: (0, i)),
            pl.BlockSpec((_H, _H + 8), lambda i: (0, 0)),
            pl.BlockSpec((_H, 1), lambda i: (0, 0)),
        ],
        out_specs=pl.BlockSpec((1, 1, _BLK), lambda i: (i, 0, 0)),
        out_shape=jax.ShapeDtypeStruct(
            (_N_PAD // _BLK, 1, _BLK), jnp.float32),
    )(z1, posT8, tabT2, wo)


@functools.lru_cache(maxsize=None)
def _sc_scatter():
    npw = 128

    @functools.partial(
        pl.kernel,
        out_type=jax.ShapeDtypeStruct((_NMOL,), jnp.float32),
        mesh=plsc.VectorSubcoreMesh(
            core_axis_name="c", subcore_axis_name="s",
            num_cores=1, num_subcores=_W),
        scratch_types=[
            pltpu.VMEM((_RPW * 128,), jnp.float32),
            pltpu.VMEM((_RPW * 128,), jnp.int32),
            pltpu.VMEM((16 * _NMOL,), jnp.float32),
            pltpu.VMEM((_NMOL,), jnp.float32),
            pltpu.VMEM((16, npw), jnp.float32),
            pltpu.VMEM((npw,), jnp.float32),
            pltpu.VMEM_SHARED((_W, _NMOL), jnp.float32),
            pltpu.SemaphoreType.DMA,
        ],
        compiler_params=pltpu.CompilerParams(needs_layout_passes=False),
    )
    def body(y_hbm, b_hbm, out_hbm, y_v, idx_v, acc, red, colblk, fin,
             stage, sem):
        wid = lax.axis_index("s")
        base = wid * (_RPW * 128)
        d1 = pltpu.async_copy(y_hbm.at[pl.ds(base, _RPW * 128)], y_v, sem)
        d2 = pltpu.async_copy(b_hbm.at[pl.ds(base, _RPW * 128)], idx_v,
                              sem)

        zero16 = jnp.zeros((16,), jnp.float32)

        def _zstore(i, c):
            for u in range(16):
                acc[pl.ds(i * 256 + u * 16, 16)] = zero16
            return c
        lax.fori_loop(0, _NMOL // 16, _zstore, 0)
        d1.wait()
        d2.wait()

        laneoff = lax.iota(jnp.int32, 16) * _NMOL
        full = jnp.ones((16,), jnp.bool_)

        def _row(r, c):
            addrs = [idx_v[pl.ds(r * 128 + cc * 16, 16)] + laneoff
                     for cc in range(8)]
            valss = [y_v[pl.ds(r * 128 + cc * 16, 16)] for cc in range(8)]
            for cc in range(8):
                plsc.addupdate_scatter(acc, [addrs[cc]], valss[cc],
                                       mask=full)
            return c
        lax.fori_loop(0, _RPW, _row, 0)

        def _lred(j, c):
            for u in range(2):
                o = j * 32 + u * 16
                s = acc[pl.ds(o, 16)]
                for l in range(1, 16):
                    s = s + acc[pl.ds(l * _NMOL + o, 16)]
                red[pl.ds(o, 16)] = s
            return c
        lax.fori_loop(0, _NMOL // 32, _lred, 0)

        pltpu.sync_copy(red, stage.at[wid])
        plsc.subcore_barrier()

        @pl.when(wid < _NMOL // npw)
        def _():
            pltpu.sync_copy(stage.at[:, pl.ds(wid * npw, npw)], colblk)
            for k in range(npw // 16):
                s = colblk[0, pl.ds(k * 16, 16)]
                for r in range(1, 16):
                    s = s + colblk[r, pl.ds(k * 16, 16)]
                fin[pl.ds(k * 16, 16)] = s
            pltpu.sync_copy(fin, out_hbm.at[pl.ds(wid * npw, npw)])

    return body


def kernel(z, pos, batch, embed, Wp, Wo):
    n = z.shape[0]
    pad = _N_PAD - n
    posT8 = jnp.pad(pos.astype(jnp.bfloat16).T,
                    ((0, 5), (0, pad)))
    b1 = jnp.pad(batch, (0, pad))
    tabT2 = jnp.concatenate(
        [jnp.pad(embed, ((0, _H - embed.shape[0]), (0, 0))).T,
         Wp.T, jnp.zeros((_H, 5), jnp.float32)],
        axis=1).astype(jnp.bfloat16)

    y = _tc_stage(z, posT8, tabT2, Wo)
    out = _sc_scatter()(y.reshape(_N_PAD), b1)
    return out.reshape(_NMOL, 1)
